# baseline (speedup 1.0000x reference)
import jax
import jax.numpy as jnp
from jax.experimental import pallas as pl
from jax.experimental.pallas import tpu as pltpu

B = 8192
D = 128
G = 128
H = 4
NEG = 0.2

BB = 4096
HG = H * G
NL = 16


def _dotT(A, X, dA, dX):
    return jax.lax.dot_general(A, X, (((dA,), (dX,)), ((), ())),
                               preferred_element_type=jnp.float32)


def _mask_src():
    r = jax.lax.broadcasted_iota(jnp.int32, (HG, NL), 0) // G
    c = jax.lax.broadcasted_iota(jnp.int32, (HG, NL), 1)
    return jnp.where((c % H == r) & (c < 3 * H), 1.0, 0.0)


def _mask_dst():
    r = jax.lax.broadcasted_iota(jnp.int32, (HG, 3 * NL), 0) // G
    c = jax.lax.broadcasted_iota(jnp.int32, (HG, 3 * NL), 1)
    blk = c // NL
    cc = c % NL
    return jnp.where((cc == blk * H + r) & (cc < 3 * H), 1.0, 0.0)


def _attend(h, lg, b_col):
    dl = lg[0][NL:, :] + lg[1][NL:, :] + lg[2][NL:, :]
    ex = []
    for i in range(3):
        L = lg[i][0:NL, :] + dl
        ex.append(jnp.where(L > 0, L, NEG * L))
    m = jnp.maximum(jnp.maximum(ex[0], ex[1]), ex[2])
    ex = [jnp.exp(v - m) for v in ex]
    inv = (1.0 / H) / (ex[0] + ex[1] + ex[2] + 1e-16)
    a1 = ex[1] * inv
    a2 = ex[2] * inv
    hc = [[h[i][k * G:(k + 1) * G, :] for k in range(H)] for i in range(3)]
    d1 = [hc[1][k] - hc[0][k] for k in range(H)]
    d2 = [hc[2][k] - hc[0][k] for k in range(H)]
    s0 = hc[0][0] + hc[0][1] + hc[0][2] + hc[0][3]
    sb = s0 * (1.0 / H) + b_col
    outs = []
    for j in range(3):
        acc = None
        for k in range(H):
            c = j * H + k
            t = a1[c:c + 1, :] * d1[k] + a2[c:c + 1, :] * d2[k]
            acc = t if acc is None else acc + t
        outs.append(acc + sb)
    return outs


def kernel(text_features, audio_features, video_features, type_emb,
           W0, att_src0, att_dst0, b0, W1, att_src1, att_dst1, b1, Wout, bout):
    as0 = att_src0.reshape(HG, 1)
    ad0 = att_dst0.reshape(HG, 1)
    as1 = att_src1.reshape(HG, 1)
    ad1 = att_dst1.reshape(HG, 1)
    b0r = b0.reshape(1, G)
    b1r = b1.reshape(1, G)
    boutr = bout.reshape(1, D)

    grid = (B // BB,)
    feat_spec = pl.BlockSpec((BB, D), lambda i: (i, 0))
    full = lambda shape: pl.BlockSpec(shape, lambda i: (0,) * len(shape))

    def body(t_ref, a_ref, v_ref, te_ref,
             W0_ref, as0_ref, ad0_ref, b0_ref,
             W1_ref, as1_ref, ad1_ref, b1_ref,
             Wout_ref, bout_ref, out_ref,
             lg0_ref, lg1_ref, bc_ref):
        @pl.when(pl.program_id(0) == 0)
        def _prep():
            msrc = _mask_src()
            mdst = _mask_dst()
            for (W_ref, as_ref, ad_ref, lg_ref) in (
                    (W0_ref, as0_ref, ad0_ref, lg0_ref),
                    (W1_ref, as1_ref, ad1_ref, lg1_ref)):
                Ws = jnp.dot(W_ref[:], msrc * as_ref[:],
                             preferred_element_type=jnp.float32)
                Wd = jnp.dot(W_ref[:], mdst * ad_ref[:],
                             preferred_element_type=jnp.float32)
                for i in range(3):
                    lg_ref[:, 2 * NL * i:2 * NL * i + NL] = Ws
                    lg_ref[:, 2 * NL * i + NL:2 * NL * (i + 1)] = (
                        Wd[:, NL * i:NL * (i + 1)])
            eye = jnp.where(
                jax.lax.broadcasted_iota(jnp.int32, (G, G), 0)
                == jax.lax.broadcasted_iota(jnp.int32, (G, G), 1), 1.0, 0.0)
            bc_ref[:, 0:1] = _dotT(eye, b0_ref[:], 0, 1)
            bc_ref[:, 1:2] = _dotT(eye, b1_ref[:], 0, 1)

        xs = [t_ref[:] + te_ref[0:1, :],
              a_ref[:] + te_ref[1:2, :],
              v_ref[:] + te_ref[2:3, :]]
        h1 = [_dotT(W0_ref[:], xs[i], 0, 1) for i in range(3)]
        lgs1 = [_dotT(lg0_ref[:, 2 * NL * i:2 * NL * (i + 1)], xs[i], 0, 1)
                for i in range(3)]
        ys = [jnp.maximum(y, 0.0) for y in _attend(h1, lgs1, bc_ref[:, 0:1])]
        h2 = [_dotT(W1_ref[:], ys[i], 0, 0) for i in range(3)]
        lgs2 = [_dotT(lg1_ref[:, 2 * NL * i:2 * NL * (i + 1)], ys[i], 0, 0)
                for i in range(3)]
        zs = [jnp.maximum(z, 0.0) for z in _attend(h2, lgs2, bc_ref[:, 1:2])]
        pooled = (zs[0] + zs[1] + zs[2]) * (1.0 / 3.0)
        out_ref[:] = _dotT(pooled, Wout_ref[:], 0, 0) + bout_ref[:]

    return pl.pallas_call(
        body,
        grid=grid,
        in_specs=[
            feat_spec, feat_spec, feat_spec,
            full((3, D)),
            full((D, HG)), full((HG, 1)), full((HG, 1)), full((1, G)),
            full((G, HG)), full((HG, 1)), full((HG, 1)), full((1, G)),
            full((G, D)), full((1, D)),
        ],
        out_specs=pl.BlockSpec((BB, D), lambda i: (i, 0)),
        out_shape=jax.ShapeDtypeStruct((B, D), jnp.float32),
        scratch_shapes=[
            pltpu.VMEM((D, 6 * NL), jnp.float32),
            pltpu.VMEM((G, 6 * NL), jnp.float32),
            pltpu.VMEM((G, 128), jnp.float32),
        ],
        compiler_params=pltpu.CompilerParams(
            dimension_semantics=("arbitrary",)),
    )(text_features, audio_features, video_features, type_emb,
      W0, as0, ad0, b0r,
      W1, as1, ad1, b1r,
      Wout, boutr)


# speedup vs baseline: 1.1564x; 1.1564x over previous
import jax
import jax.numpy as jnp
from jax.experimental import pallas as pl
from jax.experimental.pallas import tpu as pltpu

B = 8192
D = 128
G = 128
H = 4
NEG = 0.2

BB = 2048
HG = H * G
NL = 16


def _dotT(A, X, dA, dX):
    return jax.lax.dot_general(A, X, (((dA,), (dX,)), ((), ())),
                               preferred_element_type=jnp.float32)


def _mask_src():
    r = jax.lax.broadcasted_iota(jnp.int32, (HG, NL), 0) // G
    c = jax.lax.broadcasted_iota(jnp.int32, (HG, NL), 1)
    return jnp.where((c % H == r) & (c < 3 * H), 1.0, 0.0)


def _mask_dst():
    r = jax.lax.broadcasted_iota(jnp.int32, (HG, 3 * NL), 0) // G
    c = jax.lax.broadcasted_iota(jnp.int32, (HG, 3 * NL), 1)
    blk = c // NL
    cc = c % NL
    return jnp.where((cc == blk * H + r) & (cc < 3 * H), 1.0, 0.0)


def _attend(d1f, d2f, sh, lg, b_col):
    dl = lg[0][NL:, :] + lg[1][NL:, :] + lg[2][NL:, :]
    ex = []
    for i in range(3):
        L = lg[i][0:NL, :] + dl
        ex.append(jnp.where(L > 0, L, NEG * L))
    m = jnp.maximum(jnp.maximum(ex[0], ex[1]), ex[2])
    ex = [jnp.exp(v - m) for v in ex]
    inv = (1.0 / H) / (ex[0] + ex[1] + ex[2] + 1e-16)
    a1 = ex[1] * inv
    a2 = ex[2] * inv
    d1 = [d1f[k * G:(k + 1) * G, :] for k in range(H)]
    d2 = [d2f[k * G:(k + 1) * G, :] for k in range(H)]
    sb = sh + b_col
    outs = []
    for j in range(3):
        acc = None
        for k in range(H):
            c = j * H + k
            t = a1[c:c + 1, :] * d1[k] + a2[c:c + 1, :] * d2[k]
            acc = t if acc is None else acc + t
        outs.append(acc + sb)
    return outs


def kernel(text_features, audio_features, video_features, type_emb,
           W0, att_src0, att_dst0, b0, W1, att_src1, att_dst1, b1, Wout, bout):
    as0 = att_src0.reshape(HG, 1)
    ad0 = att_dst0.reshape(HG, 1)
    as1 = att_src1.reshape(HG, 1)
    ad1 = att_dst1.reshape(HG, 1)
    b0r = b0.reshape(1, G)
    b1r = b1.reshape(1, G)
    boutr = bout.reshape(1, D)

    grid = (B // BB,)
    feat_spec = pl.BlockSpec((BB, D), lambda i: (i, 0))
    full = lambda shape: pl.BlockSpec(shape, lambda i: (0,) * len(shape))

    def body(t_ref, a_ref, v_ref, te_ref,
             W0_ref, as0_ref, ad0_ref, b0_ref,
             W1_ref, as1_ref, ad1_ref, b1_ref,
             Wout_ref, bout_ref, out_ref,
             lg0_ref, lg1_ref, bc_ref, ws0_ref, ws1_ref):
        @pl.when(pl.program_id(0) == 0)
        def _prep():
            msrc = _mask_src()
            mdst = _mask_dst()
            for (W_ref, as_ref, ad_ref, lg_ref) in (
                    (W0_ref, as0_ref, ad0_ref, lg0_ref),
                    (W1_ref, as1_ref, ad1_ref, lg1_ref)):
                Ws = jnp.dot(W_ref[:], msrc * as_ref[:],
                             preferred_element_type=jnp.float32)
                Wd = jnp.dot(W_ref[:], mdst * ad_ref[:],
                             preferred_element_type=jnp.float32)
                for i in range(3):
                    lg_ref[:, 2 * NL * i:2 * NL * i + NL] = Ws
                    lg_ref[:, 2 * NL * i + NL:2 * NL * (i + 1)] = (
                        Wd[:, NL * i:NL * (i + 1)])
            eye = jnp.where(
                jax.lax.broadcasted_iota(jnp.int32, (G, G), 0)
                == jax.lax.broadcasted_iota(jnp.int32, (G, G), 1), 1.0, 0.0)
            bc_ref[:, 0:1] = _dotT(eye, b0_ref[:], 0, 1)
            bc_ref[:, 1:2] = _dotT(eye, b1_ref[:], 0, 1)
            for (W_ref, ws_ref) in ((W0_ref, ws0_ref), (W1_ref, ws1_ref)):
                ws_ref[:] = (W_ref[:, 0:G] + W_ref[:, G:2 * G]
                             + W_ref[:, 2 * G:3 * G]
                             + W_ref[:, 3 * G:4 * G]) * (1.0 / H)

        xs = [t_ref[:] + te_ref[0:1, :],
              a_ref[:] + te_ref[1:2, :],
              v_ref[:] + te_ref[2:3, :]]
        d1f = _dotT(W0_ref[:], xs[1] - xs[0], 0, 1)
        d2f = _dotT(W0_ref[:], xs[2] - xs[0], 0, 1)
        sh1 = _dotT(ws0_ref[:], xs[0], 0, 1)
        lgs1 = [_dotT(lg0_ref[:, 2 * NL * i:2 * NL * (i + 1)], xs[i], 0, 1)
                for i in range(3)]
        ys = [jnp.maximum(y, 0.0)
              for y in _attend(d1f, d2f, sh1, lgs1, bc_ref[:, 0:1])]
        e1f = _dotT(W1_ref[:], ys[1] - ys[0], 0, 0)
        e2f = _dotT(W1_ref[:], ys[2] - ys[0], 0, 0)
        sh2 = _dotT(ws1_ref[:], ys[0], 0, 0)
        lgs2 = [_dotT(lg1_ref[:, 2 * NL * i:2 * NL * (i + 1)], ys[i], 0, 0)
                for i in range(3)]
        zs = [jnp.maximum(z, 0.0)
              for z in _attend(e1f, e2f, sh2, lgs2, bc_ref[:, 1:2])]
        pooled = (zs[0] + zs[1] + zs[2]) * (1.0 / 3.0)
        out_ref[:] = _dotT(pooled, Wout_ref[:], 0, 0) + bout_ref[:]

    return pl.pallas_call(
        body,
        grid=grid,
        in_specs=[
            feat_spec, feat_spec, feat_spec,
            full((3, D)),
            full((D, HG)), full((HG, 1)), full((HG, 1)), full((1, G)),
            full((G, HG)), full((HG, 1)), full((HG, 1)), full((1, G)),
            full((G, D)), full((1, D)),
        ],
        out_specs=pl.BlockSpec((BB, D), lambda i: (i, 0)),
        out_shape=jax.ShapeDtypeStruct((B, D), jnp.float32),
        scratch_shapes=[
            pltpu.VMEM((D, 6 * NL), jnp.float32),
            pltpu.VMEM((G, 6 * NL), jnp.float32),
            pltpu.VMEM((G, 128), jnp.float32),
            pltpu.VMEM((D, G), jnp.float32),
            pltpu.VMEM((G, G), jnp.float32),
        ],
        compiler_params=pltpu.CompilerParams(
            dimension_semantics=("arbitrary",)),
    )(text_features, audio_features, video_features, type_emb,
      W0, as0, ad0, b0r,
      W1, as1, ad1, b1r,
      Wout, boutr)
